# baseline (speedup 1.0000x reference)
import functools

import jax
import jax.numpy as jnp
from jax import lax
from jax.experimental import pallas as pl
from jax.experimental.pallas import tpu as pltpu
from jax.experimental.pallas import tpu_sc as plsc

B, S, D = 4, 2048, 1024
NC, NS = 2, 16
NW = NC * NS
RW = S // NW
CH = 8
NCH = RW // CH
CHW = CH * D
LANES = 16
VECS = CHW // LANES
UNROLL = 4

_mesh = plsc.VectorSubcoreMesh(core_axis_name="c", subcore_axis_name="s")


@functools.partial(
    pl.kernel,
    mesh=_mesh,
    out_type=jax.ShapeDtypeStruct((B, S * D), jnp.float32),
    scratch_types=[
        pltpu.VMEM((2, B, CHW), jnp.float32),
        pltpu.VMEM((2, CHW), jnp.float32),
        pltpu.SemaphoreType.DMA,
        pltpu.SemaphoreType.DMA,
        pltpu.SemaphoreType.DMA,
        pltpu.SemaphoreType.DMA,
    ],
)
def _pos_add(x_hbm, emb_hbm, out_hbm, x_buf, emb_buf,
             in_sem0, in_sem1, out_sem0, out_sem1):
    wid = lax.axis_index("s") * NC + lax.axis_index("c")
    base = wid * RW * D

    in_sems = (in_sem0, in_sem1)
    out_sems = (out_sem0, out_sem1)

    def load(j, slot):
        off = pl.multiple_of(base + j * CHW, 8)
        hs = [pltpu.async_copy(emb_hbm.at[pl.ds(off, CHW)],
                               emb_buf.at[slot], in_sems[slot])]
        for b in range(B):
            hs.append(pltpu.async_copy(x_hbm.at[b, pl.ds(off, CHW)],
                                       x_buf.at[slot, b], in_sems[slot]))
        return hs

    def store(j, slot):
        off = pl.multiple_of(base + j * CHW, 8)
        return [pltpu.async_copy(x_buf.at[slot, b],
                                 out_hbm.at[b, pl.ds(off, CHW)],
                                 out_sems[slot])
                for b in range(B)]

    def compute(slot):
        def step(i, carry):
            off = i * (LANES * UNROLL)
            for u in range(UNROLL):
                o = off + u * LANES
                v = emb_buf[slot, pl.ds(o, LANES)]
                for b in range(B):
                    plsc.addupdate(x_buf.at[slot, b, pl.ds(o, LANES)], v)
            return carry
        lax.fori_loop(0, VECS // UNROLL, step, 0)

    pend_out = {0: [], 1: []}
    pend_in = {0: load(0, 0), 1: []}
    for j in range(NCH):
        slot = j & 1
        nxt = (j + 1) & 1
        if j + 1 < NCH:
            for h in pend_out[nxt]:
                h.wait()
            pend_in[nxt] = load(j + 1, nxt)
        for h in pend_in[slot]:
            h.wait()
        compute(slot)
        pend_out[slot] = store(j, slot)
    for sl in (0, 1):
        for h in pend_out[sl]:
            h.wait()


def kernel(x, emb):
    xf = x.reshape(B, S * D)
    ef = emb.reshape(S * D)
    out = _pos_add(xf, ef)
    return out.reshape(B, S, D)


# speedup vs baseline: 1.9545x; 1.9545x over previous
import functools

import jax
import jax.numpy as jnp
from jax import lax
from jax.experimental import pallas as pl
from jax.experimental.pallas import tpu as pltpu
from jax.experimental.pallas import tpu_sc as plsc

B, S, D = 4, 2048, 1024
NC, NS = 2, 16
NW = NC * NS
RW = S // NW
CH = 8
NCH = RW // CH
LANES = 16
VPR = D // LANES

_mesh = plsc.VectorSubcoreMesh(core_axis_name="c", subcore_axis_name="s")


@functools.partial(
    pl.kernel,
    mesh=_mesh,
    out_type=jax.ShapeDtypeStruct((B, S, D), jnp.float32),
    scratch_types=[
        pltpu.VMEM((2, B, CH, D), jnp.float32),
        pltpu.VMEM((2, CH, D), jnp.float32),
        pltpu.SemaphoreType.DMA,
        pltpu.SemaphoreType.DMA,
        pltpu.SemaphoreType.DMA,
        pltpu.SemaphoreType.DMA,
    ],
)
def _pos_add(x_hbm, emb_hbm, out_hbm, x_buf, emb_buf,
             in_sem0, in_sem1, out_sem0, out_sem1):
    wid = lax.axis_index("s") * NC + lax.axis_index("c")
    row0 = wid * RW

    in_sems = (in_sem0, in_sem1)
    out_sems = (out_sem0, out_sem1)

    def load(j, slot):
        rows = pl.multiple_of(row0 + j * CH, CH)
        hs = [pltpu.async_copy(emb_hbm.at[pl.ds(rows, CH)],
                               emb_buf.at[slot], in_sems[slot])]
        for b in range(B):
            hs.append(pltpu.async_copy(x_hbm.at[b, pl.ds(rows, CH)],
                                       x_buf.at[slot, b], in_sems[slot]))
        return hs

    def store(j, slot):
        rows = pl.multiple_of(row0 + j * CH, CH)
        return [pltpu.async_copy(x_buf.at[slot, b],
                                 out_hbm.at[b, pl.ds(rows, CH)],
                                 out_sems[slot])
                for b in range(B)]

    def compute(slot):
        def step(i, carry):
            o = i * LANES
            for r in range(CH):
                v = emb_buf[slot, r, pl.ds(o, LANES)]
                for b in range(B):
                    plsc.addupdate(x_buf.at[slot, b, r, pl.ds(o, LANES)], v)
            return carry
        lax.fori_loop(0, VPR, step, 0)

    pend_out = {0: [], 1: []}
    pend_in = {0: load(0, 0), 1: []}
    for j in range(NCH):
        slot = j & 1
        nxt = (j + 1) & 1
        if j + 1 < NCH:
            for h in pend_out[nxt]:
                h.wait()
            pend_in[nxt] = load(j + 1, nxt)
        for h in pend_in[slot]:
            h.wait()
        compute(slot)
        pend_out[slot] = store(j, slot)
    for sl in (0, 1):
        for h in pend_out[sl]:
            h.wait()


def kernel(x, emb):
    return _pos_add(x, emb)


# speedup vs baseline: 2.1934x; 1.1222x over previous
import functools

import jax
import jax.numpy as jnp
from jax import lax
from jax.experimental import pallas as pl
from jax.experimental.pallas import tpu as pltpu
from jax.experimental.pallas import tpu_sc as plsc

B, S, D = 4, 2048, 1024
NC, NS = 2, 16
NW = NC * NS
RW = S // NW
CH = 8
NCH = RW // CH
LANES = 16
VPR = D // LANES

_mesh = plsc.VectorSubcoreMesh(core_axis_name="c", subcore_axis_name="s")


@functools.partial(
    pl.kernel,
    mesh=_mesh,
    out_type=jax.ShapeDtypeStruct((B, S, D), jnp.float32),
    scratch_types=[
        pltpu.VMEM((2, B, CH, D), jnp.float32),
        pltpu.VMEM((2, CH, D), jnp.float32),
        pltpu.SemaphoreType.DMA,
        pltpu.SemaphoreType.DMA,
        pltpu.SemaphoreType.DMA,
        pltpu.SemaphoreType.DMA,
    ],
)
def _pos_add(x_hbm, emb_hbm, out_hbm, x_buf, emb_buf,
             in_sem0, in_sem1, out_sem0, out_sem1):
    wid = lax.axis_index("s") * NC + lax.axis_index("c")
    row0 = wid * RW

    in_sems = (in_sem0, in_sem1)
    out_sems = (out_sem0, out_sem1)

    def load(j, slot):
        rows = pl.multiple_of(row0 + j * CH, CH)
        hs = [pltpu.async_copy(emb_hbm.at[pl.ds(rows, CH)],
                               emb_buf.at[slot], in_sems[slot])]
        for b in range(B):
            hs.append(pltpu.async_copy(x_hbm.at[b, pl.ds(rows, CH)],
                                       x_buf.at[slot, b], in_sems[slot]))
        return hs

    def store(j, slot):
        rows = pl.multiple_of(row0 + j * CH, CH)
        return [pltpu.async_copy(x_buf.at[slot, b],
                                 out_hbm.at[b, pl.ds(rows, CH)],
                                 out_sems[slot])
                for b in range(B)]

    def compute(slot):
        def step(i, carry):
            o = i * LANES
            vs = [emb_buf[slot, r, pl.ds(o, LANES)] for r in range(CH)]
            for r in range(CH):
                for b in range(B):
                    plsc.addupdate(x_buf.at[slot, b, r, pl.ds(o, LANES)],
                                   vs[r])
            return carry
        lax.fori_loop(0, VPR, step, 0)

    pend_out = {0: [], 1: []}
    pend_in = {0: load(0, 0), 1: []}
    for j in range(NCH):
        slot = j & 1
        nxt = (j + 1) & 1
        if j + 1 < NCH:
            for h in pend_out[nxt]:
                h.wait()
            pend_in[nxt] = load(j + 1, nxt)
        for h in pend_in[slot]:
            h.wait()
        compute(slot)
        pend_out[slot] = store(j, slot)
    for sl in (0, 1):
        for h in pend_out[sl]:
            h.wait()


def kernel(x, emb):
    return _pos_add(x, emb)


# speedup vs baseline: 2.2392x; 1.0209x over previous
import functools

import jax
import jax.numpy as jnp
from jax import lax
from jax.experimental import pallas as pl
from jax.experimental.pallas import tpu as pltpu
from jax.experimental.pallas import tpu_sc as plsc

B, S, D = 4, 2048, 1024
NC, NS = 2, 16
NW = NC * NS
RW = S // NW
CH = 8
NCH = RW // CH
LANES = 16
VPR = D // LANES

_mesh = plsc.VectorSubcoreMesh(core_axis_name="c", subcore_axis_name="s")


@functools.partial(
    pl.kernel,
    mesh=_mesh,
    out_type=jax.ShapeDtypeStruct((B, S, D), jnp.float32),
    scratch_types=[
        pltpu.VMEM((3, B, CH, D), jnp.float32),
        pltpu.VMEM((3, CH, D), jnp.float32),
        pltpu.SemaphoreType.DMA,
        pltpu.SemaphoreType.DMA,
        pltpu.SemaphoreType.DMA,
        pltpu.SemaphoreType.DMA,
        pltpu.SemaphoreType.DMA,
        pltpu.SemaphoreType.DMA,
    ],
)
def _pos_add(x_hbm, emb_hbm, out_hbm, x_buf, emb_buf,
             in_sem0, in_sem1, in_sem2, out_sem0, out_sem1, out_sem2):
    wid = lax.axis_index("s") * NC + lax.axis_index("c")
    row0 = wid * RW

    in_sems = (in_sem0, in_sem1, in_sem2)
    out_sems = (out_sem0, out_sem1, out_sem2)

    def load(j, slot):
        rows = pl.multiple_of(row0 + j * CH, CH)
        return [pltpu.async_copy(emb_hbm.at[pl.ds(rows, CH)],
                                 emb_buf.at[slot], in_sems[slot]),
                pltpu.async_copy(x_hbm.at[:, pl.ds(rows, CH)],
                                 x_buf.at[slot], in_sems[slot])]

    def store(j, slot):
        rows = pl.multiple_of(row0 + j * CH, CH)
        return [pltpu.async_copy(x_buf.at[slot],
                                 out_hbm.at[:, pl.ds(rows, CH)],
                                 out_sems[slot])]

    def compute(slot):
        def step(i, carry):
            o = i * LANES
            vs = [emb_buf[slot, r, pl.ds(o, LANES)] for r in range(CH)]
            for r in range(CH):
                for b in range(B):
                    plsc.addupdate(x_buf.at[slot, b, r, pl.ds(o, LANES)],
                                   vs[r])
            return carry
        lax.fori_loop(0, VPR, step, 0)

    NSLOT = 3
    pend_out = {s: [] for s in range(NSLOT)}
    pend_in = {s: [] for s in range(NSLOT)}
    pend_in[0] = load(0, 0)
    pend_in[1] = load(1, 1)
    for j in range(NCH):
        slot = j % NSLOT
        nxt = (j + 2) % NSLOT
        if j + 2 < NCH:
            for h in pend_out[nxt]:
                h.wait()
            pend_in[nxt] = load(j + 2, nxt)
        for h in pend_in[slot]:
            h.wait()
        compute(slot)
        pend_out[slot] = store(j, slot)
    for sl in range(NSLOT):
        for h in pend_out[sl]:
            h.wait()


def kernel(x, emb):
    return _pos_add(x, emb)


# speedup vs baseline: 2.2394x; 1.0001x over previous
import functools

import jax
import jax.numpy as jnp
from jax import lax
from jax.experimental import pallas as pl
from jax.experimental.pallas import tpu as pltpu
from jax.experimental.pallas import tpu_sc as plsc

B, S, D = 4, 2048, 1024
NC, NS = 2, 16
NW = NC * NS
RW = S // NW
CH = 8
NCH = RW // CH
LANES = 16
VPR = D // LANES

_mesh = plsc.VectorSubcoreMesh(core_axis_name="c", subcore_axis_name="s")


@functools.partial(
    pl.kernel,
    mesh=_mesh,
    out_type=jax.ShapeDtypeStruct((B, S, D), jnp.float32),
    scratch_types=[
        pltpu.VMEM((3, B, CH, D), jnp.float32),
        pltpu.VMEM((3, CH, D), jnp.float32),
        pltpu.SemaphoreType.DMA,
        pltpu.SemaphoreType.DMA,
        pltpu.SemaphoreType.DMA,
        pltpu.SemaphoreType.DMA,
        pltpu.SemaphoreType.DMA,
        pltpu.SemaphoreType.DMA,
        pltpu.SMEM((8,), jnp.int32),
    ],
)
def _pos_add(x_hbm, emb_hbm, out_hbm, x_buf, emb_buf,
             in_sem0, in_sem1, in_sem2, out_sem0, out_sem1, out_sem2,
             smem_scratch):
    wid = lax.axis_index("s") * NC + lax.axis_index("c")
    row0 = wid * RW

    in_sems = (in_sem0, in_sem1, in_sem2)
    out_sems = (out_sem0, out_sem1, out_sem2)

    def load(j, slot):
        rows = pl.multiple_of(row0 + j * CH, CH)
        return [pltpu.async_copy(emb_hbm.at[pl.ds(rows, CH)],
                                 emb_buf.at[slot], in_sems[slot]),
                pltpu.async_copy(x_hbm.at[:, pl.ds(rows, CH)],
                                 x_buf.at[slot], in_sems[slot])]

    def store(j, slot):
        rows = pl.multiple_of(row0 + j * CH, CH)
        return [pltpu.async_copy(x_buf.at[slot],
                                 out_hbm.at[:, pl.ds(rows, CH)],
                                 out_sems[slot])]

    def compute(slot):
        def step(i, carry):
            o = i * LANES
            vs = [emb_buf[slot, r, pl.ds(o, LANES)] for r in range(CH)]
            for r in range(CH):
                for b in range(B):
                    plsc.addupdate(x_buf.at[slot, b, r, pl.ds(o, LANES)],
                                   vs[r])
            return carry
        lax.fori_loop(0, VPR, step, 0)

    skew = lax.fori_loop(0, (wid % 2) * 400, lambda i, c: c + i, 0)
    smem_scratch[0] = skew

    NSLOT = 3
    pend_out = {s: [] for s in range(NSLOT)}
    pend_in = {s: [] for s in range(NSLOT)}
    pend_in[0] = load(0, 0)
    pend_in[1] = load(1, 1)
    for j in range(NCH):
        slot = j % NSLOT
        nxt = (j + 2) % NSLOT
        if j + 2 < NCH:
            for h in pend_out[nxt]:
                h.wait()
            pend_in[nxt] = load(j + 2, nxt)
        for h in pend_in[slot]:
            h.wait()
        compute(slot)
        pend_out[slot] = store(j, slot)
    for sl in range(NSLOT):
        for h in pend_out[sl]:
            h.wait()


def kernel(x, emb):
    return _pos_add(x, emb)


# speedup vs baseline: 2.2515x; 1.0054x over previous
import functools

import jax
import jax.numpy as jnp
from jax import lax
from jax.experimental import pallas as pl
from jax.experimental.pallas import tpu as pltpu
from jax.experimental.pallas import tpu_sc as plsc

B, S, D = 4, 2048, 1024
NC, NS = 2, 16
NW = NC * NS
RW = S // NW
CH = 8
NCH = RW // CH
LANES = 16
VPR = D // LANES

_mesh = plsc.VectorSubcoreMesh(core_axis_name="c", subcore_axis_name="s")


@functools.partial(
    pl.kernel,
    mesh=_mesh,
    out_type=jax.ShapeDtypeStruct((B, S, D), jnp.float32),
    scratch_types=[
        pltpu.VMEM((3, B, CH, D), jnp.float32),
        pltpu.VMEM((3, CH, D), jnp.float32),
        pltpu.SemaphoreType.DMA,
        pltpu.SemaphoreType.DMA,
        pltpu.SemaphoreType.DMA,
        pltpu.SemaphoreType.DMA,
        pltpu.SemaphoreType.DMA,
        pltpu.SemaphoreType.DMA,
        pltpu.SMEM((8,), jnp.int32),
    ],
)
def _pos_add(x_hbm, emb_hbm, out_hbm, x_buf, emb_buf,
             in_sem0, in_sem1, in_sem2, out_sem0, out_sem1, out_sem2,
             smem_scratch):
    wid = lax.axis_index("s") * NC + lax.axis_index("c")
    row0 = wid * RW

    in_sems = (in_sem0, in_sem1, in_sem2)
    out_sems = (out_sem0, out_sem1, out_sem2)

    def load(j, slot):
        rows = pl.multiple_of(row0 + j * CH, CH)
        return [pltpu.async_copy(emb_hbm.at[pl.ds(rows, CH)],
                                 emb_buf.at[slot], in_sems[slot]),
                pltpu.async_copy(x_hbm.at[:, pl.ds(rows, CH)],
                                 x_buf.at[slot], in_sems[slot])]

    def store(j, slot):
        rows = pl.multiple_of(row0 + j * CH, CH)
        return [pltpu.async_copy(x_buf.at[slot],
                                 out_hbm.at[:, pl.ds(rows, CH)],
                                 out_sems[slot])]

    def compute(slot):
        def step(i, carry):
            o = i * LANES
            vs = [emb_buf[slot, r, pl.ds(o, LANES)] for r in range(CH)]
            for r in range(CH):
                for b in range(B):
                    plsc.addupdate(x_buf.at[slot, b, r, pl.ds(o, LANES)],
                                   vs[r])
            return carry
        lax.fori_loop(0, VPR, step, 0)

    skew = lax.fori_loop(0, (wid % 2) * 1600, lambda i, c: c + i, 0)
    smem_scratch[0] = skew

    NSLOT = 3
    pend_out = {s: [] for s in range(NSLOT)}
    pend_in = {s: [] for s in range(NSLOT)}
    pend_in[0] = load(0, 0)
    pend_in[1] = load(1, 1)
    for j in range(NCH):
        slot = j % NSLOT
        nxt = (j + 2) % NSLOT
        if j + 2 < NCH:
            for h in pend_out[nxt]:
                h.wait()
            pend_in[nxt] = load(j + 2, nxt)
        for h in pend_in[slot]:
            h.wait()
        compute(slot)
        pend_out[slot] = store(j, slot)
    for sl in range(NSLOT):
        for h in pend_out[sl]:
            h.wait()


def kernel(x, emb):
    return _pos_add(x, emb)
